# baseline (speedup 1.0000x reference)
import jax
import jax.numpy as jnp
from jax.experimental import pallas as pl


def _gate_body(e_ref, o_ref):
    e = e_ref[...]
    fi = (e > 0.5).astype(jnp.int32)
    fh = fi.reshape(-1, 128)
    f4 = jnp.repeat(fh, 2, axis=0)
    par = jax.lax.broadcasted_iota(jnp.int32, f4.shape, 0) & 1
    o_ref[...] = f4 ^ (1 - par)


_GRID = 1
_RB = 1024 // _GRID

_gate_tc = pl.pallas_call(
    _gate_body,
    grid=(_GRID,),
    in_specs=[pl.BlockSpec((_RB, 256), lambda i: (i, 0))],
    out_specs=pl.BlockSpec((4 * _RB, 128), lambda i: (i, 0)),
    out_shape=jax.ShapeDtypeStruct((4096, 128), jnp.int32),
)


def kernel(entropy):
    e2d = entropy.transpose(1, 2, 0).reshape(1024, 256)
    out2d = _gate_tc(e2d)
    o = out2d.reshape(32, 32, 2, 2, 128)
    o = o.transpose(2, 4, 0, 1, 3)
    return o.reshape(256, 32, 32, 2)


# speedup vs baseline: 1.7627x; 1.7627x over previous
import jax
import jax.numpy as jnp
from jax.experimental import pallas as pl
from jax.experimental.pallas import tpu as pltpu


def _gate_body(e_ref, o_ref):
    e = e_ref[...]
    fi = (e > 0.5).astype(jnp.int32)
    fh = fi.reshape(-1, 128)
    w = (fh ^ 1) | (fh << 16)
    o_ref[...] = pltpu.bitcast(w, jnp.int16).astype(jnp.int32)


_GRID = 2
_RB = 1024 // _GRID

_gate_tc = pl.pallas_call(
    _gate_body,
    grid=(_GRID,),
    in_specs=[pl.BlockSpec((_RB, 256), lambda i: (i, 0))],
    out_specs=pl.BlockSpec((4 * _RB, 128), lambda i: (i, 0)),
    out_shape=jax.ShapeDtypeStruct((4096, 128), jnp.int32),
)


def kernel(entropy):
    e2d = entropy.transpose(1, 2, 0).reshape(1024, 256)
    out2d = _gate_tc(e2d)
    o = out2d.reshape(32, 32, 2, 2, 128)
    o = o.transpose(2, 4, 0, 1, 3)
    return o.reshape(256, 32, 32, 2)
